# baseline (speedup 1.0000x reference)
import jax
import jax.numpy as jnp
from jax.experimental import pallas as pl
from jax.experimental.pallas import tpu as pltpu

E = 8
R = 64
D = 2048
ER = E * R
SCALING = 4.0

BM = 1024
SUB = 4


def _fused_dense_kernel(x_ref, wg_ref, wa_ref, wb_ref, o_ref):
    wg = wg_ref[...]
    wa = wa_ref[...]
    wb = wb_ref[...]
    hb = BM // SUB
    for s in range(SUB):
        x = x_ref[s * hb:(s + 1) * hb, :]
        logits = jax.lax.dot_general(x, wg, (((1,), (1,)), ((), ())),
                                     preferred_element_type=jnp.float32)
        h = jax.lax.dot_general(x, wa, (((1,), (1,)), ((), ())),
                                preferred_element_type=jnp.float32)
        maxv = jnp.max(logits, axis=1, keepdims=True)
        denom = jnp.sum(jnp.exp(logits - maxv), axis=1, keepdims=True)
        coef = SCALING / denom
        eidx = jax.lax.broadcasted_iota(jnp.int32, logits.shape, 1)
        gate = jnp.min(jnp.where(logits >= maxv, eidx, E), axis=1,
                       keepdims=True)
        col_e = jax.lax.broadcasted_iota(jnp.int32, (hb, ER), 1) // R
        cmat = jnp.where(col_e == gate, coef, 0.0)
        o_ref[s * hb:(s + 1) * hb, :] = jax.lax.dot_general(
            h * cmat, wb, (((1,), (0,)), ((), ())),
            preferred_element_type=jnp.float32)


@jax.jit
def kernel(x, Wg, WA, WB):
    bsz, seq, d = x.shape
    T = bsz * seq
    xf = x.reshape(T, d)
    WA_all = WA.reshape(ER, D)
    WB_stack = WB.transpose(0, 2, 1).reshape(ER, D)

    out = pl.pallas_call(
        _fused_dense_kernel,
        grid=(T // BM,),
        in_specs=[
            pl.BlockSpec((BM, D), lambda i: (i, 0)),
            pl.BlockSpec((E, D), lambda i: (0, 0)),
            pl.BlockSpec((ER, D), lambda i: (0, 0)),
            pl.BlockSpec((ER, D), lambda i: (0, 0)),
        ],
        out_specs=pl.BlockSpec((BM, D), lambda i: (i, 0)),
        out_shape=jax.ShapeDtypeStruct((T, D), jnp.float32),
        compiler_params=pltpu.CompilerParams(dimension_semantics=("parallel",)),
    )(xf, Wg, WA_all, WB_stack)
    return out.reshape(bsz, seq, d)


# speedup vs baseline: 1.0870x; 1.0870x over previous
import jax
import jax.numpy as jnp
from jax.experimental import pallas as pl
from jax.experimental.pallas import tpu as pltpu

E = 8
R = 64
D = 2048
ER = E * R
SCALING = 4.0

BM = 1024
SUB = 4


def _fused_dense_kernel(x_ref, wg_ref, wa_ref, wb_ref, o_ref):
    wg = wg_ref[...]
    wa = wa_ref[...]
    wb = wb_ref[...]
    hb = BM // SUB
    xs = [x_ref[s * hb:(s + 1) * hb, :] for s in range(SUB)]
    logits = [jax.lax.dot_general(x, wg, (((1,), (1,)), ((), ())),
                                  preferred_element_type=jnp.float32)
              for x in xs]
    cmats = []
    for lg in logits:
        maxv = jnp.max(lg, axis=1, keepdims=True)
        denom = jnp.sum(jnp.exp(lg - maxv), axis=1, keepdims=True)
        coef = SCALING / denom
        eidx = jax.lax.broadcasted_iota(jnp.int32, lg.shape, 1)
        gate = jnp.min(jnp.where(lg >= maxv, eidx, E), axis=1, keepdims=True)
        col_e = jax.lax.broadcasted_iota(jnp.int32, (hb, ER), 1) // R
        cmats.append(jnp.where(col_e == gate, coef, 0.0))
    hs = [jax.lax.dot_general(x, wa, (((1,), (1,)), ((), ())),
                              preferred_element_type=jnp.float32)
          for x in xs]
    for s in range(SUB):
        o_ref[s * hb:(s + 1) * hb, :] = jax.lax.dot_general(
            hs[s] * cmats[s], wb, (((1,), (0,)), ((), ())),
            preferred_element_type=jnp.float32)


@jax.jit
def kernel(x, Wg, WA, WB):
    bsz, seq, d = x.shape
    T = bsz * seq
    xf = x.reshape(T, d)
    WA_all = WA.reshape(ER, D)
    WB_stack = WB.transpose(0, 2, 1).reshape(ER, D)

    out = pl.pallas_call(
        _fused_dense_kernel,
        grid=(T // BM,),
        in_specs=[
            pl.BlockSpec((BM, D), lambda i: (i, 0)),
            pl.BlockSpec((E, D), lambda i: (0, 0)),
            pl.BlockSpec((ER, D), lambda i: (0, 0)),
            pl.BlockSpec((ER, D), lambda i: (0, 0)),
        ],
        out_specs=pl.BlockSpec((BM, D), lambda i: (i, 0)),
        out_shape=jax.ShapeDtypeStruct((T, D), jnp.float32),
        compiler_params=pltpu.CompilerParams(dimension_semantics=("parallel",)),
    )(xf, Wg, WA_all, WB_stack)
    return out.reshape(bsz, seq, d)
